# baseline (speedup 1.0000x reference)
import functools

import jax
import jax.numpy as jnp
from jax import lax
from jax.experimental import pallas as pl
from jax.experimental.pallas import tpu as pltpu
from jax.experimental.pallas import tpu_sc as plsc

N_NODES = 10000
N_EDGES = 320000
N_PAD = 10240
E_PAD = 327680
N_EROWS = E_PAD // 128
IN_DIM = 128
HID_DIM = 128
NUM_CLASSES = 32

NC = 2
NS = 16
CHUNK = 128

_MESH = dict(core_axis_name="c", subcore_axis_name="s", num_cores=NC,
             num_subcores=NS)


def _deg_body(e2d, zvec, degs_out, degd_out, deg_sh, idx_all, ones_v, sem):
    c = lax.axis_index("c")
    s = lax.axis_index("s")
    for i in range(8):
        ones_v[pl.ds(i * 16, 16)] = jnp.ones((16,), jnp.float32)
    pltpu.sync_copy(zvec.at[pl.ds(s * 640, 640)], deg_sh.at[pl.ds(s * 640, 640)])
    rows_per_tile = N_EROWS // NS
    pltpu.sync_copy(e2d.at[pl.ds(c * N_EROWS + s * rows_per_tile,
                                 rows_per_tile)], idx_all)
    plsc.subcore_barrier()

    def outer(g, carry):
        for b in range(8):
            pltpu.async_copy(ones_v, deg_sh.at[idx_all.at[g * 8 + b]], sem,
                             add=True)
        for b in range(8):
            pltpu.make_async_copy(zvec.at[pl.ds(0, CHUNK)], ones_v, sem).wait()
        return carry

    lax.fori_loop(0, rows_per_tile // 8, outer, 0)
    plsc.subcore_barrier()

    @pl.when(c == 0)
    def _():
        pltpu.sync_copy(deg_sh.at[pl.ds(s * 640, 640)],
                        degs_out.at[pl.ds(s * 640, 640)])

    @pl.when(c == 1)
    def _():
        pltpu.sync_copy(deg_sh.at[pl.ds(s * 640, 640)],
                        degd_out.at[pl.ds(s * 640, 640)])


_deg_kernel = functools.partial(
    pl.kernel,
    out_type=[jax.ShapeDtypeStruct((N_PAD,), jnp.float32),
              jax.ShapeDtypeStruct((N_PAD,), jnp.float32)],
    mesh=plsc.VectorSubcoreMesh(**_MESH),
    scratch_types=[
        pltpu.VMEM_SHARED((N_PAD,), jnp.float32),
        pltpu.VMEM((N_EROWS // NS, CHUNK), jnp.int32),
        pltpu.VMEM((CHUNK,), jnp.float32),
        pltpu.SemaphoreType.DMA,
    ],
)(_deg_body)


CHUNKC = 64
NBUF = 4
N_CROWS = E_PAD // CHUNKC


def _edge_body(e2d, x, nd, zrows, zvec, aggp, cp, agg_sh, c_sh,
               isrc, idst, rows, ndv, sem_g, sem_n, sem_s):
    c = lax.axis_index("c")
    s = lax.axis_index("s")
    w = c * NS + s
    rows_per_w = N_CROWS // (NC * NS)
    pltpu.sync_copy(zrows.at[pl.ds(s * 640, 640)],
                    agg_sh.at[pl.ds(s * 640, 640)])
    pltpu.sync_copy(zvec.at[pl.ds(s * 640, 640)], c_sh.at[pl.ds(s * 640, 640)])
    ph_rows = rows_per_w // 4

    def drain_scatter():
        pltpu.make_async_copy(x.at[pl.ds(0, CHUNKC)], rows.at[0],
                              sem_s).wait()

    for ph in range(4):
        pltpu.sync_copy(
            e2d.at[pl.ds(w * rows_per_w + ph * ph_rows, ph_rows)], isrc)
        pltpu.sync_copy(
            e2d.at[pl.ds(N_CROWS + w * rows_per_w + ph * ph_rows, ph_rows)],
            idst)
        if ph == 0:
            plsc.subcore_barrier()
        for b in range(2):
            pltpu.async_copy(x.at[isrc.at[b]], rows.at[b], sem_g)
            pltpu.async_copy(nd.at[idst.at[b]], ndv.at[b], sem_n)

        def outer(g, carry):
            for b in range(NBUF):
                j = g * NBUF + b
                pltpu.make_async_copy(x.at[pl.ds(0, CHUNKC)], rows.at[b],
                                      sem_g).wait()
                pltpu.async_copy(rows.at[b], agg_sh.at[idst.at[j]], sem_s,
                                 add=True)
                pltpu.make_async_copy(nd.at[pl.ds(0, CHUNKC)], ndv.at[b],
                                      sem_n).wait()
                pltpu.sync_copy(ndv.at[b], c_sh.at[isrc.at[j]], add=True)

                @pl.when(j >= 2)
                def _():
                    drain_scatter()

                @pl.when(j <= ph_rows - 3)
                def _():
                    bn = (b + 2) % NBUF
                    pltpu.async_copy(x.at[isrc.at[j + 2]], rows.at[bn], sem_g)
                    pltpu.async_copy(nd.at[idst.at[j + 2]], ndv.at[bn], sem_n)
            return carry

        lax.fori_loop(0, ph_rows // NBUF, outer, 0)
        drain_scatter()
        drain_scatter()
    plsc.subcore_barrier()
    pltpu.sync_copy(agg_sh.at[pl.ds(s * 640, 640)],
                    aggp.at[c, pl.ds(s * 640, 640)])
    pltpu.sync_copy(c_sh.at[pl.ds(s * 640, 640)],
                    cp.at[c, pl.ds(s * 640, 640)])


_edge_kernel = functools.partial(
    pl.kernel,
    out_type=[jax.ShapeDtypeStruct((NC, N_PAD, HID_DIM), jnp.float32),
              jax.ShapeDtypeStruct((NC, N_PAD), jnp.float32)],
    mesh=plsc.VectorSubcoreMesh(**_MESH),
    scratch_types=[
        pltpu.VMEM_SHARED((N_PAD, HID_DIM), jnp.float32),
        pltpu.VMEM_SHARED((N_PAD,), jnp.float32),
        pltpu.VMEM((N_CROWS // (NC * NS) // 4, CHUNKC), jnp.int32),
        pltpu.VMEM((N_CROWS // (NC * NS) // 4, CHUNKC), jnp.int32),
        pltpu.VMEM((NBUF, CHUNKC, HID_DIM), jnp.float32),
        pltpu.VMEM((NBUF, CHUNKC), jnp.float32),
        pltpu.SemaphoreType.DMA,
        pltpu.SemaphoreType.DMA,
        pltpu.SemaphoreType.DMA,
    ],
)(_edge_body)


def _norm_mm_body(h_ref, w1_ref, ds_ref, dd_ref, x_ref, ns_ref, nd_ref):
    ns = lax.rsqrt(jnp.maximum(ds_ref[...], 1.0))
    nd = lax.rsqrt(jnp.maximum(dd_ref[...], 1.0))
    ns_ref[...] = ns
    nd_ref[...] = nd
    y = jnp.dot(h_ref[...], w1_ref[...], preferred_element_type=jnp.float32)
    x_ref[:N_NODES] = y * ns[:N_NODES]
    x_ref[N_NODES:] = jnp.zeros((N_PAD - N_NODES, HID_DIM), jnp.float32)


def _finish_body(aggp_ref, cp_ref, ns_ref, nd_ref, b1_ref, w2_ref, b2_ref,
                 out_ref):
    agg = aggp_ref[0] + aggp_ref[1]
    h1 = jnp.maximum(agg * nd_ref[...] + b1_ref[...], 0.0)
    w = ns_ref[...] * (cp_ref[0] + cp_ref[1])
    row = lax.broadcasted_iota(jnp.int32, (N_PAD, 1), 0)
    w = jnp.where(row < N_NODES, w, 0.0)
    s = jnp.sum(h1 * w, axis=0, keepdims=True)
    out_ref[...] = (jnp.dot(s, w2_ref[...], preferred_element_type=jnp.float32)
                    * (1.0 / N_NODES) + b2_ref[...])


def kernel(h, edge_index, W1, b1, W2, b2):
    src = edge_index[0].astype(jnp.int32)
    dst = edge_index[1].astype(jnp.int32)
    pad = N_NODES + (jnp.arange(E_PAD - N_EDGES, dtype=jnp.int32)
                     % (N_PAD - N_NODES))
    flat = jnp.concatenate([src, pad, dst, pad])
    e2d = flat.reshape(2 * N_EROWS, CHUNK)
    e2dc = flat.reshape(2 * N_CROWS, CHUNKC)
    zvec = jnp.zeros((N_PAD,), jnp.float32)
    zrows = jnp.zeros((N_PAD, HID_DIM), jnp.float32)

    deg_src, deg_dst = _deg_kernel(e2d, zvec)

    x, ns_col, nd_col = pl.pallas_call(
        _norm_mm_body,
        out_shape=[
            jax.ShapeDtypeStruct((N_PAD, HID_DIM), jnp.float32),
            jax.ShapeDtypeStruct((N_PAD, 1), jnp.float32),
            jax.ShapeDtypeStruct((N_PAD, 1), jnp.float32),
        ],
    )(h, W1, deg_src[:, None], deg_dst[:, None])

    aggp, cp = _edge_kernel(e2dc, x, nd_col.reshape(N_PAD), zrows, zvec)

    out = pl.pallas_call(
        _finish_body,
        out_shape=jax.ShapeDtypeStruct((1, NUM_CLASSES), jnp.float32),
    )(aggp, cp[:, :, None], ns_col, nd_col, b1[None, :], W2, b2[None, :])
    return out


# speedup vs baseline: 1.0641x; 1.0641x over previous
import functools

import jax
import jax.numpy as jnp
from jax import lax
from jax.experimental import pallas as pl
from jax.experimental.pallas import tpu as pltpu
from jax.experimental.pallas import tpu_sc as plsc

N_NODES = 10000
N_EDGES = 320000
N_PAD = 10240
E_PAD = 327680
N_EROWS = E_PAD // 128
IN_DIM = 128
HID_DIM = 128
NUM_CLASSES = 32

NC = 2
NS = 16
CHUNK = 128

_MESH = dict(core_axis_name="c", subcore_axis_name="s", num_cores=NC,
             num_subcores=NS)


def _deg_body(e2d, zvec, degs_out, degd_out, deg_sh, idx_all, ones_v, sem):
    c = lax.axis_index("c")
    s = lax.axis_index("s")
    for i in range(8):
        ones_v[pl.ds(i * 16, 16)] = jnp.ones((16,), jnp.float32)
    pltpu.sync_copy(zvec.at[pl.ds(s * 640, 640)], deg_sh.at[pl.ds(s * 640, 640)])
    rows_per_tile = N_EROWS // NS
    pltpu.sync_copy(e2d.at[pl.ds(c * N_EROWS + s * rows_per_tile,
                                 rows_per_tile)], idx_all)
    plsc.subcore_barrier()

    def outer(g, carry):
        for b in range(8):
            pltpu.async_copy(ones_v, deg_sh.at[idx_all.at[g * 8 + b]], sem,
                             add=True)
        for b in range(8):
            pltpu.make_async_copy(zvec.at[pl.ds(0, CHUNK)], ones_v, sem).wait()
        return carry

    lax.fori_loop(0, rows_per_tile // 8, outer, 0)
    plsc.subcore_barrier()

    @pl.when(c == 0)
    def _():
        pltpu.sync_copy(deg_sh.at[pl.ds(s * 640, 640)],
                        degs_out.at[pl.ds(s * 640, 640)])

    @pl.when(c == 1)
    def _():
        pltpu.sync_copy(deg_sh.at[pl.ds(s * 640, 640)],
                        degd_out.at[pl.ds(s * 640, 640)])


_deg_kernel = functools.partial(
    pl.kernel,
    out_type=[jax.ShapeDtypeStruct((N_PAD,), jnp.float32),
              jax.ShapeDtypeStruct((N_PAD,), jnp.float32)],
    mesh=plsc.VectorSubcoreMesh(**_MESH),
    scratch_types=[
        pltpu.VMEM_SHARED((N_PAD,), jnp.float32),
        pltpu.VMEM((N_EROWS // NS, CHUNK), jnp.int32),
        pltpu.VMEM((CHUNK,), jnp.float32),
        pltpu.SemaphoreType.DMA,
    ],
)(_deg_body)


def _edge_body(e2d, x, nd, zrows, zvec, aggp, cp, agg_sh, c_sh,
               isrc, idst, rows, ndv, sem_g, sem_n):
    c = lax.axis_index("c")
    s = lax.axis_index("s")
    w = c * NS + s
    rows_per_w = N_EROWS // (NC * NS)
    pltpu.sync_copy(zrows.at[pl.ds(s * 640, 640)],
                    agg_sh.at[pl.ds(s * 640, 640)])
    pltpu.sync_copy(zvec.at[pl.ds(s * 640, 640)], c_sh.at[pl.ds(s * 640, 640)])
    ph_rows = rows_per_w // 2

    for ph in range(2):
        pltpu.sync_copy(
            e2d.at[pl.ds(w * rows_per_w + ph * ph_rows, ph_rows)], isrc)
        pltpu.sync_copy(
            e2d.at[pl.ds(N_EROWS + w * rows_per_w + ph * ph_rows, ph_rows)],
            idst)
        if ph == 0:
            plsc.subcore_barrier()
        for b in range(2):
            pltpu.async_copy(x.at[isrc.at[b]], rows.at[b], sem_g)
            pltpu.async_copy(nd.at[idst.at[b]], ndv.at[b], sem_n)

        def outer(g, carry):
            for b in range(2):
                j = g * 2 + b
                pltpu.make_async_copy(x.at[pl.ds(0, CHUNK)], rows.at[b],
                                      sem_g).wait()
                pltpu.sync_copy(rows.at[b], agg_sh.at[idst.at[j]], add=True)
                pltpu.make_async_copy(nd.at[pl.ds(0, CHUNK)], ndv.at[b],
                                      sem_n).wait()
                pltpu.sync_copy(ndv.at[b], c_sh.at[isrc.at[j]], add=True)

                @pl.when(g < ph_rows // 2 - 1)
                def _():
                    pltpu.async_copy(x.at[isrc.at[j + 2]], rows.at[b], sem_g)
                    pltpu.async_copy(nd.at[idst.at[j + 2]], ndv.at[b], sem_n)
            return carry

        lax.fori_loop(0, ph_rows // 2, outer, 0)
    plsc.subcore_barrier()
    pltpu.sync_copy(agg_sh.at[pl.ds(s * 640, 640)],
                    aggp.at[c, pl.ds(s * 640, 640)])
    pltpu.sync_copy(c_sh.at[pl.ds(s * 640, 640)],
                    cp.at[c, pl.ds(s * 640, 640)])


_edge_kernel = functools.partial(
    pl.kernel,
    out_type=[jax.ShapeDtypeStruct((NC, N_PAD, HID_DIM), jnp.float32),
              jax.ShapeDtypeStruct((NC, N_PAD), jnp.float32)],
    mesh=plsc.VectorSubcoreMesh(**_MESH),
    scratch_types=[
        pltpu.VMEM_SHARED((N_PAD, HID_DIM), jnp.float32),
        pltpu.VMEM_SHARED((N_PAD,), jnp.float32),
        pltpu.VMEM((N_EROWS // (NC * NS) // 2, CHUNK), jnp.int32),
        pltpu.VMEM((N_EROWS // (NC * NS) // 2, CHUNK), jnp.int32),
        pltpu.VMEM((2, CHUNK, HID_DIM), jnp.float32),
        pltpu.VMEM((2, CHUNK), jnp.float32),
        pltpu.SemaphoreType.DMA,
        pltpu.SemaphoreType.DMA,
    ],
)(_edge_body)


def _norm_mm_body(h_ref, w1_ref, ds_ref, dd_ref, x_ref, ns_ref, nd_ref):
    ns = lax.rsqrt(jnp.maximum(ds_ref[...], 1.0))
    nd = lax.rsqrt(jnp.maximum(dd_ref[...], 1.0))
    ns_ref[...] = ns
    nd_ref[...] = nd
    y = jnp.dot(h_ref[...], w1_ref[...], preferred_element_type=jnp.float32)
    x_ref[:N_NODES] = y * ns[:N_NODES]
    x_ref[N_NODES:] = jnp.zeros((N_PAD - N_NODES, HID_DIM), jnp.float32)


DBLK = 1024
DGRID = N_PAD // DBLK


def _finish_body(aggp_ref, cp_ref, ns_ref, nd_ref, b1_ref, w2_ref, b2_ref,
                 out_ref, acc_ref):
    i = pl.program_id(0)

    @pl.when(i == 0)
    def _():
        acc_ref[...] = jnp.zeros((1, HID_DIM), jnp.float32)

    agg = aggp_ref[0] + aggp_ref[1]
    h1 = jnp.maximum(agg * nd_ref[...] + b1_ref[...], 0.0)
    w = ns_ref[...] * (cp_ref[0] + cp_ref[1])
    row = lax.broadcasted_iota(jnp.int32, (DBLK, 1), 0) + i * DBLK
    w = jnp.where(row < N_NODES, w, 0.0)
    acc_ref[...] += jnp.sum(h1 * w, axis=0, keepdims=True)

    @pl.when(i == DGRID - 1)
    def _():
        out_ref[...] = (jnp.dot(acc_ref[...], w2_ref[...],
                                preferred_element_type=jnp.float32)
                        * (1.0 / N_NODES) + b2_ref[...])


def kernel(h, edge_index, W1, b1, W2, b2):
    src = edge_index[0].astype(jnp.int32)
    dst = edge_index[1].astype(jnp.int32)
    pad = N_NODES + (jnp.arange(E_PAD - N_EDGES, dtype=jnp.int32)
                     % (N_PAD - N_NODES))
    e2d = jnp.concatenate([src, pad, dst, pad]).reshape(2 * N_EROWS, CHUNK)
    zvec = jnp.zeros((N_PAD,), jnp.float32)
    zrows = jnp.zeros((N_PAD, HID_DIM), jnp.float32)

    deg_src, deg_dst = _deg_kernel(e2d, zvec)

    x, ns_col, nd_col = pl.pallas_call(
        _norm_mm_body,
        out_shape=[
            jax.ShapeDtypeStruct((N_PAD, HID_DIM), jnp.float32),
            jax.ShapeDtypeStruct((N_PAD, 1), jnp.float32),
            jax.ShapeDtypeStruct((N_PAD, 1), jnp.float32),
        ],
    )(h, W1, deg_src[:, None], deg_dst[:, None])

    aggp, cp = _edge_kernel(e2d, x, nd_col.reshape(N_PAD), zrows, zvec)

    out = pl.pallas_call(
        _finish_body,
        grid=(DGRID,),
        in_specs=[
            pl.BlockSpec((NC, DBLK, HID_DIM), lambda i: (0, i, 0)),
            pl.BlockSpec((NC, DBLK, 1), lambda i: (0, i, 0)),
            pl.BlockSpec((DBLK, 1), lambda i: (i, 0)),
            pl.BlockSpec((DBLK, 1), lambda i: (i, 0)),
            pl.BlockSpec((1, HID_DIM), lambda i: (0, 0)),
            pl.BlockSpec((HID_DIM, NUM_CLASSES), lambda i: (0, 0)),
            pl.BlockSpec((1, NUM_CLASSES), lambda i: (0, 0)),
        ],
        out_specs=pl.BlockSpec((1, NUM_CLASSES), lambda i: (0, 0)),
        out_shape=jax.ShapeDtypeStruct((1, NUM_CLASSES), jnp.float32),
        scratch_shapes=[pltpu.VMEM((1, HID_DIM), jnp.float32)],
    )(aggp, cp[:, :, None], ns_col, nd_col, b1[None, :], W2, b2[None, :])
    return out
